# baseline (speedup 1.0000x reference)
import jax
import jax.numpy as jnp
from jax.experimental import pallas as pl

N = 8192
BR = 512


def _bt_block(a_rows_ref, a_cols_ref, out_ref):
    d = a_rows_ref[...] - a_cols_ref[...]
    out_ref[...] = 0.5 * jnp.tanh(0.5 * d) + 0.5


def kernel(ability):
    a_rows = ability.reshape(N, 1)
    a_cols = ability.reshape(1, N)
    return pl.pallas_call(
        _bt_block,
        grid=(N // BR,),
        in_specs=[
            pl.BlockSpec((BR, 1), lambda i: (i, 0)),
            pl.BlockSpec((1, N), lambda i: (0, 0)),
        ],
        out_specs=pl.BlockSpec((BR, N), lambda i: (i, 0)),
        out_shape=jax.ShapeDtypeStruct((N, N), jnp.float32),
    )(a_rows, a_cols)
